# baseline (speedup 1.0000x reference)
import functools
import jax
import jax.numpy as jnp
from jax import lax
from jax.experimental import pallas as pl
from jax.experimental.pallas import tpu as pltpu
from jax.experimental.pallas import tpu_sc as plsc

N = 10000
E = 320000
DIN = 128
HID = 64
H = 8
F1 = 512
DOUT = 128

NC = 2
NS = 16
NW = NC * NS

NHALF = N // 2
NACC = NHALF + 8
EC = E // NW
KA = 400
KB = 800
BN = 1000

NP16 = 640
M1 = 100000
SPAN1 = 3136
M2 = NP16 * 16
SPAN2 = M2 // NW


def _iota16():
    return lax.broadcasted_iota(jnp.int32, (16,), 0)


def _t1_body(x_ref, w_ref, asrc_ref, adst_ref, we_ref, ae_ref,
             h_ref, tabS_ref, tabD_ref, c_ref):
    h = jnp.dot(x_ref[...], w_ref[...], preferred_element_type=jnp.float32)
    h_ref[...] = h
    row = lax.broadcasted_iota(jnp.int32, (F1, 16), 0)
    col = lax.broadcasted_iota(jnp.int32, (F1, 16), 1)
    blk = (row // HID) == (col % H)
    blk8 = blk[:, 0:8]
    As = jnp.where(blk8, asrc_ref[...].T, 0.0)
    Ad = jnp.where(blk8, adst_ref[...].T, 0.0)
    as1 = jnp.dot(h, As, preferred_element_type=jnp.float32)
    ad1 = jnp.dot(h, Ad, preferred_element_type=jnp.float32)
    tabS_ref[...] = jnp.concatenate([as1, as1], axis=1)
    tabD_ref[...] = jnp.concatenate([ad1, ad1], axis=1)
    pe = we_ref[...] * ae_ref[...]
    M16 = jnp.where(blk, 1.0, 0.0)
    c_ref[...] = jnp.dot(pe, M16, preferred_element_type=jnp.float32)


def _t1(x, W1, a_src1, a_dst1, We1, ae1):
    return pl.pallas_call(
        _t1_body,
        grid=(N // BN,),
        in_specs=[
            pl.BlockSpec((BN, DIN), lambda i: (i, 0)),
            pl.BlockSpec((DIN, F1), lambda i: (0, 0)),
            pl.BlockSpec((1, F1), lambda i: (0, 0)),
            pl.BlockSpec((1, F1), lambda i: (0, 0)),
            pl.BlockSpec((1, F1), lambda i: (0, 0)),
            pl.BlockSpec((1, F1), lambda i: (0, 0)),
        ],
        out_specs=[
            pl.BlockSpec((BN, F1), lambda i: (i, 0)),
            pl.BlockSpec((BN, 16), lambda i: (i, 0)),
            pl.BlockSpec((BN, 16), lambda i: (i, 0)),
            pl.BlockSpec((1, 16), lambda i: (0, 0)),
        ],
        out_shape=[
            jax.ShapeDtypeStruct((N, F1), jnp.float32),
            jax.ShapeDtypeStruct((N, 16), jnp.float32),
            jax.ShapeDtypeStruct((N, 16), jnp.float32),
            jax.ShapeDtypeStruct((1, 16), jnp.float32),
        ],
    )(x, W1, a_src1, a_dst1, We1, ae1)


def _sa1(src, dst, w, tabS, tabD, c1arr):
    mesh = plsc.VectorSubcoreMesh(core_axis_name="c", subcore_axis_name="s")

    @functools.partial(
        pl.kernel,
        out_type=[
            jax.ShapeDtypeStruct((H * E + KB,), jnp.float32),
            jax.ShapeDtypeStruct((NW, 32 * SPAN1), jnp.float32),
        ],
        mesh=mesh,
        compiler_params=pltpu.CompilerParams(use_tc_tiling_on_sc=False, needs_layout_passes=False),
        scratch_types=[
            pltpu.VMEM((KA,), jnp.int32),
            pltpu.VMEM((KA,), jnp.int32),
            pltpu.VMEM((KA,), jnp.float32),
            pltpu.VMEM((KA, 16), jnp.float32),
            pltpu.VMEM((KA, 16), jnp.float32),
            pltpu.VMEM((H * KA,), jnp.float32),
            pltpu.VMEM((N * 8,), jnp.float32),
            pltpu.VMEM((N * 2,), jnp.float32),
            pltpu.VMEM((16,), jnp.float32),
            pltpu.SemaphoreType.DMA,
            pltpu.SemaphoreType.DMA,
        ],
    )
    def sa1(src_hbm, dst_hbm, w_hbm, tabS_hbm, tabD_hbm, c1_hbm,
            p_hbm, part_hbm,
            src_v, dst_v, w_v, rowsS, rowsD, pbuf, accP, accWC,
            c1_v, sem1, sem2):
        wid = lax.axis_index("s") * NC + lax.axis_index("c")
        iota = _iota16()
        zero16 = jnp.zeros((16,), jnp.float32)
        lane_lt8 = iota < 8
        lane_lt2 = iota < 2
        iota_ka = iota * KA

        pltpu.sync_copy(c1_hbm, c1_v)
        c1v = c1_v[...]

        def zP(i, _):
            accP[pl.ds(i * 16, 16)] = zero16
            return 0
        lax.fori_loop(0, (N * 8) // 16, zP, 0)

        def zWC(i, _):
            accWC[pl.ds(i * 16, 16)] = zero16
            return 0
        lax.fori_loop(0, (N * 2) // 16, zWC, 0)

        base0 = wid * EC

        def chunk_body(ci, _):
            base = base0 + ci * KA
            pltpu.sync_copy(src_hbm.at[pl.ds(base, KA)], src_v)
            pltpu.sync_copy(dst_hbm.at[pl.ds(base, KA)], dst_v)
            pltpu.sync_copy(w_hbm.at[pl.ds(base, KA)], w_v)
            d1 = pltpu.async_copy(tabS_hbm.at[src_v], rowsS, sem1)
            d2 = pltpu.async_copy(tabD_hbm.at[dst_v], rowsD, sem2)
            d1.wait()
            d2.wait()

            def edge_grp(j, _):
                jf = jnp.full((16,), j, jnp.int32)
                dv16 = plsc.load_gather(dst_v, [jf])
                wsp = plsc.load_gather(w_v, [jf])
                vas = plsc.load_gather(rowsS, [jf, iota])
                vad = plsc.load_gather(rowsD, [jf, iota])
                z = vas + vad + wsp * c1v
                z = jnp.maximum(z, 0.2 * z)
                p16 = jnp.exp(z)
                plsc.store_scatter(pbuf, [iota_ka + j], p16, mask=lane_lt8)
                plsc.addupdate_scatter(accP, [dv16 * 8 + iota], p16,
                                       mask=lane_lt8)
                plsc.addupdate_scatter(accWC, [dv16 * 2 + iota],
                                       jnp.where(iota == 0, wsp, 1.0),
                                       mask=lane_lt2)
                return 0
            lax.fori_loop(0, KA, edge_grp, 0)
            for h in range(H):
                pltpu.sync_copy(pbuf.at[pl.ds(h * KA, KA)],
                                p_hbm.at[pl.ds(h * E + base, KA)])
            return 0
        lax.fori_loop(0, EC // KA, chunk_body, 0)
        pltpu.sync_copy(accP, part_hbm.at[wid, pl.ds(0, N * 8)])
        pltpu.sync_copy(accWC, part_hbm.at[wid, pl.ds(N * 8, N * 2)])

    return sa1(src, dst, w, tabS, tabD, c1arr)


def _reduce_partials(part, mprow, span):
    mesh = plsc.VectorSubcoreMesh(core_axis_name="c", subcore_axis_name="s")

    @functools.partial(
        pl.kernel,
        out_type=jax.ShapeDtypeStruct((NW * span,), jnp.float32),
        mesh=mesh,
        compiler_params=pltpu.CompilerParams(use_tc_tiling_on_sc=False, needs_layout_passes=False),
        scratch_types=[
            pltpu.VMEM((span,), jnp.float32),
            pltpu.VMEM((span,), jnp.float32),
        ],
    )
    def red_k(part_hbm, out_hbm, buf, accv):
        wid = lax.axis_index("s") * NC + lax.axis_index("c")
        off = wid * span
        zero16 = jnp.zeros((16,), jnp.float32)

        def zbody(i, _):
            accv[pl.ds(i * 16, 16)] = zero16
            return 0
        lax.fori_loop(0, span // 16, zbody, 0)

        def tbody(t, _):
            pltpu.sync_copy(part_hbm.at[pl.ds(t * mprow + off, span)], buf)

            def abody(i, _):
                sl = pl.ds(i * 16, 16)
                accv[sl] = accv[sl] + buf[sl]
                return 0
            lax.fori_loop(0, span // 16, abody, 0)
            return 0
        lax.fori_loop(0, NW, tbody, 0)
        pltpu.sync_copy(accv, out_hbm.at[pl.ds(off, span)])

    return red_k(part)


def _sb1(src, dst, p1, h1v):
    mesh = plsc.VectorSubcoreMesh(core_axis_name="c", subcore_axis_name="s")

    @functools.partial(
        pl.kernel,
        out_type=jax.ShapeDtypeStruct((2, NHALF, NW, 16), jnp.float32),
        mesh=mesh,
        compiler_params=pltpu.CompilerParams(use_tc_tiling_on_sc=False, needs_layout_passes=False),
        scratch_types=[
            pltpu.VMEM((KB,), jnp.int32),
            pltpu.VMEM((KB,), jnp.int32),
            pltpu.VMEM((KB,), jnp.float32),
            pltpu.VMEM((KB,), jnp.float32),
            pltpu.VMEM((KB,), jnp.int32),
            pltpu.VMEM((KB,), jnp.int32),
            pltpu.VMEM((KB,), jnp.int32),
            pltpu.VMEM((KB,), jnp.int32),
            pltpu.VMEM((KB, 16), jnp.float32),
            pltpu.VMEM((KB, 16), jnp.float32),
            pltpu.VMEM((NACC, 16), jnp.float32),
            pltpu.SemaphoreType.DMA,
            pltpu.SemaphoreType.DMA,
        ],
    )
    def sb1(src_hbm, dst_hbm, p_hbm, h1_hbm, out_hbm,
            src_v, dst_v, p0, p1_, idx0, idx1, dl0, dl1, rows0, rows1,
            acc, sem0, sem1):
        wid = lax.axis_index("s") * NC + lax.axis_index("c")
        head = wid // 4
        iota = _iota16()
        zero16 = jnp.zeros((16,), jnp.float32)
        NCH = E // KB

        for half in range(2):
            def zbody(i, _):
                acc[i] = zero16
                return 0
            lax.fori_loop(0, NACC, zbody, 0)

            def load_prep(ci, p_b, idx_b, dl_b):
                base = ci * KB
                pltpu.sync_copy(src_hbm.at[pl.ds(base, KB)], src_v)
                pltpu.sync_copy(dst_hbm.at[pl.ds(base, KB)], dst_v)
                pltpu.sync_copy(p_hbm.at[pl.ds(head * E + base, KB)], p_b)

                def prep(g, _):
                    sl = pl.ds(g * 16, 16)
                    sv = src_v[sl]
                    idx_b[sl] = sv * NW + wid
                    dv = dst_v[sl] - half * NHALF
                    m = (dv >= 0) & (dv < NHALF)
                    dl_b[sl] = jnp.where(m, dv, NHALF)
                    return 0
                lax.fori_loop(0, KB // 16, prep, 0)

            def compute(rows_b, p_b, dl_b):
                def edge_grp(g, _):
                    gs16 = jnp.full((16,), g * 16, jnp.int32)
                    for u in range(16):
                        jf = gs16 + u
                        psp = plsc.load_gather(p_b, [jf])
                        dsp = plsc.load_gather(dl_b, [jf])
                        rowv = rows_b[g * 16 + u]
                        plsc.addupdate_scatter(acc, [dsp, iota], rowv * psp)
                    return 0
                lax.fori_loop(0, KB // 16, edge_grp, 0)

            load_prep(0, p0, idx0, dl0)
            pltpu.async_copy(h1_hbm.at[idx0], rows0, sem0)

            def pair(k, _):
                c0 = 2 * k
                load_prep(c0 + 1, p1_, idx1, dl1)
                pltpu.async_copy(h1_hbm.at[idx1], rows1, sem1)
                pltpu.make_async_copy(h1_hbm.at[idx0], rows0, sem0).wait()
                compute(rows0, p0, dl0)
                load_prep(c0 + 2, p0, idx0, dl0)
                pltpu.async_copy(h1_hbm.at[idx0], rows0, sem0)
                pltpu.make_async_copy(h1_hbm.at[idx1], rows1, sem1).wait()
                compute(rows1, p1_, dl1)
                return 0
            lax.fori_loop(0, NCH // 2, pair, 0)
            pltpu.make_async_copy(h1_hbm.at[idx0], rows0, sem0).wait()
            pltpu.sync_copy(acc.at[pl.ds(0, NHALF), :],
                            out_hbm.at[half, :, wid, :])

    return sb1(src, dst, p1, h1v)


def _t2_body(acc_ref, psum_ref, wsum_ref, cnt_ref, tabS_ref, tabD_ref,
             c_ref, h_ref, b_ref, w2_ref,
             asrc2_ref, adst2_ref, we2_ref, ae2_ref,
             h2_ref, tab2S_ref, tab2D_ref, c2_ref):
    psum = psum_ref[...]
    la = wsum_ref[...] / jnp.maximum(cnt_ref[...], 1.0)
    as1 = tabS_ref[...][:, 0:8]
    ad1 = tabD_ref[...][:, 0:8]
    c1 = c_ref[...][:, 0:8]
    z = as1 + ad1 + la * c1
    z = jnp.maximum(z, 0.2 * z)
    pself = jnp.exp(z)
    ssum = psum + pself + 1e-16
    h1b = h_ref[...]
    accb = acc_ref[...]
    rrow = lax.broadcasted_iota(jnp.int32, (H, F1), 0)
    rcol = lax.broadcasted_iota(jnp.int32, (H, F1), 1)
    R = jnp.where(rrow == (rcol // HID), 1.0, 0.0)
    pw = jnp.dot(pself, R, preferred_element_type=jnp.float32)
    sw = jnp.dot(ssum, R, preferred_element_type=jnp.float32)
    h1o = (accb + pw * h1b) / sw + b_ref[...]
    act = jnp.where(h1o > 0, h1o, jnp.exp(jnp.minimum(h1o, 0.0)) - 1.0)
    h2 = jnp.dot(act, w2_ref[...], preferred_element_type=jnp.float32)
    h2_ref[...] = h2
    as2 = jnp.sum(h2 * asrc2_ref[...], axis=1, keepdims=True)
    ad2 = jnp.sum(h2 * adst2_ref[...], axis=1, keepdims=True)
    ones16 = jnp.ones((1, 16), jnp.float32)
    tab2S_ref[...] = as2 * ones16
    tab2D_ref[...] = ad2 * ones16
    c2 = jnp.sum(we2_ref[...] * ae2_ref[...], axis=1, keepdims=True)
    c2_ref[...] = c2 * ones16


def _t2(acc1, psum, wsum, cnt, tabS, tabD, c1arr, h1, b1, W2,
        a_src2, a_dst2, We2, ae2):
    return pl.pallas_call(
        _t2_body,
        grid=(N // BN,),
        in_specs=[
            pl.BlockSpec((BN, F1), lambda i: (i, 0)),
            pl.BlockSpec((BN, 8), lambda i: (i, 0)),
            pl.BlockSpec((BN, 1), lambda i: (i, 0)),
            pl.BlockSpec((BN, 1), lambda i: (i, 0)),
            pl.BlockSpec((BN, 16), lambda i: (i, 0)),
            pl.BlockSpec((BN, 16), lambda i: (i, 0)),
            pl.BlockSpec((1, 16), lambda i: (0, 0)),
            pl.BlockSpec((BN, F1), lambda i: (i, 0)),
            pl.BlockSpec((1, F1), lambda i: (0, 0)),
            pl.BlockSpec((F1, DOUT), lambda i: (0, 0)),
            pl.BlockSpec((1, DOUT), lambda i: (0, 0)),
            pl.BlockSpec((1, DOUT), lambda i: (0, 0)),
            pl.BlockSpec((1, DOUT), lambda i: (0, 0)),
            pl.BlockSpec((1, DOUT), lambda i: (0, 0)),
        ],
        out_specs=[
            pl.BlockSpec((BN, DOUT), lambda i: (i, 0)),
            pl.BlockSpec((BN, 16), lambda i: (i, 0)),
            pl.BlockSpec((BN, 16), lambda i: (i, 0)),
            pl.BlockSpec((1, 16), lambda i: (0, 0)),
        ],
        out_shape=[
            jax.ShapeDtypeStruct((N, DOUT), jnp.float32),
            jax.ShapeDtypeStruct((N, 16), jnp.float32),
            jax.ShapeDtypeStruct((N, 16), jnp.float32),
            jax.ShapeDtypeStruct((1, 16), jnp.float32),
        ],
    )(acc1, psum, wsum, cnt, tabS, tabD, c1arr, h1, b1, W2,
      a_src2, a_dst2, We2, ae2)


def _sa2(src, dst, w, tab2S, tab2D, c2arr):
    mesh = plsc.VectorSubcoreMesh(core_axis_name="c", subcore_axis_name="s")

    @functools.partial(
        pl.kernel,
        out_type=[
            jax.ShapeDtypeStruct((E + KB,), jnp.float32),
            jax.ShapeDtypeStruct((NW, NP16, 16), jnp.float32),
        ],
        mesh=mesh,
        compiler_params=pltpu.CompilerParams(use_tc_tiling_on_sc=False, needs_layout_passes=False),
        scratch_types=[
            pltpu.VMEM((KA,), jnp.int32),
            pltpu.VMEM((KA,), jnp.int32),
            pltpu.VMEM((KA,), jnp.float32),
            pltpu.VMEM((KA, 16), jnp.float32),
            pltpu.VMEM((KA, 16), jnp.float32),
            pltpu.VMEM((KA,), jnp.float32),
            pltpu.VMEM((NP16, 16), jnp.float32),
            pltpu.VMEM((16,), jnp.float32),
            pltpu.SemaphoreType.DMA,
            pltpu.SemaphoreType.DMA,
        ],
    )
    def sa2(src_hbm, dst_hbm, w_hbm, tabS_hbm, tabD_hbm, c2_hbm,
            p_hbm, part_hbm,
            src_v, dst_v, w_v, rowsS, rowsD, p_v, accS, c2_v,
            sem1, sem2):
        wid = lax.axis_index("s") * NC + lax.axis_index("c")
        iota = _iota16()
        zero16 = jnp.zeros((16,), jnp.float32)
        izero = jnp.zeros((16,), jnp.int32)
        lane0 = iota == 0

        pltpu.sync_copy(c2_hbm, c2_v)
        c2v = c2_v[...]

        def zS(i, _):
            accS[i] = zero16
            return 0
        lax.fori_loop(0, NP16, zS, 0)

        base0 = wid * EC

        def chunk_body(ci, _):
            base = base0 + ci * KA
            pltpu.sync_copy(src_hbm.at[pl.ds(base, KA)], src_v)
            pltpu.sync_copy(dst_hbm.at[pl.ds(base, KA)], dst_v)
            pltpu.sync_copy(w_hbm.at[pl.ds(base, KA)], w_v)
            d1 = pltpu.async_copy(tabS_hbm.at[src_v], rowsS, sem1)
            d2 = pltpu.async_copy(tabD_hbm.at[dst_v], rowsD, sem2)
            d1.wait()
            d2.wait()

            def edge_grp(g, _):
                sl = pl.ds(g * 16, 16)
                dvec = dst_v[sl]
                wvec = w_v[sl]
                for u in range(16):
                    j = g * 16 + u
                    dv = dvec[u]
                    z = rowsS[j] + rowsD[j] + wvec[u] * c2v
                    z = jnp.maximum(z, 0.2 * z)
                    p16 = jnp.exp(z)
                    plsc.store_scatter(p_v, [izero + j], p16, mask=lane0)
                    plsc.addupdate(accS.at[dv // 16],
                                   jnp.where(iota == (dv % 16), p16, 0.0))
                return 0
            lax.fori_loop(0, KA // 16, edge_grp, 0)
            pltpu.sync_copy(p_v, p_hbm.at[pl.ds(base, KA)])
            return 0
        lax.fori_loop(0, EC // KA, chunk_body, 0)
        pltpu.sync_copy(accS, part_hbm.at[wid])

    return sa2(src, dst, w, tab2S, tab2D, c2arr)


def _sb2(src, dst, p2, h2v):
    mesh = plsc.VectorSubcoreMesh(core_axis_name="c", subcore_axis_name="s")
    EQ = E // 4

    @functools.partial(
        pl.kernel,
        out_type=jax.ShapeDtypeStruct((2, NHALF, 4, 8, 16), jnp.float32),
        mesh=mesh,
        compiler_params=pltpu.CompilerParams(use_tc_tiling_on_sc=False, needs_layout_passes=False),
        scratch_types=[
            pltpu.VMEM((KB,), jnp.int32),
            pltpu.VMEM((KB,), jnp.int32),
            pltpu.VMEM((KB,), jnp.float32),
            pltpu.VMEM((KB,), jnp.float32),
            pltpu.VMEM((KB,), jnp.int32),
            pltpu.VMEM((KB,), jnp.int32),
            pltpu.VMEM((KB,), jnp.int32),
            pltpu.VMEM((KB,), jnp.int32),
            pltpu.VMEM((KB, 16), jnp.float32),
            pltpu.VMEM((KB, 16), jnp.float32),
            pltpu.VMEM((NACC, 16), jnp.float32),
            pltpu.SemaphoreType.DMA,
            pltpu.SemaphoreType.DMA,
        ],
    )
    def sb2(src_hbm, dst_hbm, p_hbm, h2_hbm, out_hbm,
            src_v, dst_v, p0, p1_, idx0, idx1, dl0, dl1, rows0, rows1,
            acc, sem0, sem1):
        wid = lax.axis_index("s") * NC + lax.axis_index("c")
        q = wid // 8
        col = wid % 8
        iota = _iota16()
        zero16 = jnp.zeros((16,), jnp.float32)
        ebase = q * EQ
        NCH = EQ // KB

        for half in range(2):
            def zbody(i, _):
                acc[i] = zero16
                return 0
            lax.fori_loop(0, NACC, zbody, 0)

            def load_prep(ci, p_b, idx_b, dl_b):
                base = ebase + ci * KB
                pltpu.sync_copy(src_hbm.at[pl.ds(base, KB)], src_v)
                pltpu.sync_copy(dst_hbm.at[pl.ds(base, KB)], dst_v)
                pltpu.sync_copy(p_hbm.at[pl.ds(base, KB)], p_b)

                def prep(g, _):
                    sl = pl.ds(g * 16, 16)
                    sv = src_v[sl]
                    idx_b[sl] = sv * 8 + col
                    dv = dst_v[sl] - half * NHALF
                    m = (dv >= 0) & (dv < NHALF)
                    dl_b[sl] = jnp.where(m, dv, NHALF)
                    return 0
                lax.fori_loop(0, KB // 16, prep, 0)

            def compute(rows_b, p_b, dl_b):
                def edge_grp(g, _):
                    gs16 = jnp.full((16,), g * 16, jnp.int32)
                    for u in range(16):
                        jf = gs16 + u
                        psp = plsc.load_gather(p_b, [jf])
                        dsp = plsc.load_gather(dl_b, [jf])
                        rowv = rows_b[g * 16 + u]
                        plsc.addupdate_scatter(acc, [dsp, iota], rowv * psp)
                    return 0
                lax.fori_loop(0, KB // 16, edge_grp, 0)

            load_prep(0, p0, idx0, dl0)
            pltpu.async_copy(h2_hbm.at[idx0], rows0, sem0)

            def pair(k, _):
                c0 = 2 * k
                load_prep(c0 + 1, p1_, idx1, dl1)
                pltpu.async_copy(h2_hbm.at[idx1], rows1, sem1)
                pltpu.make_async_copy(h2_hbm.at[idx0], rows0, sem0).wait()
                compute(rows0, p0, dl0)
                load_prep(c0 + 2, p0, idx0, dl0)
                pltpu.async_copy(h2_hbm.at[idx0], rows0, sem0)
                pltpu.make_async_copy(h2_hbm.at[idx1], rows1, sem1).wait()
                compute(rows1, p1_, dl1)
                return 0
            lax.fori_loop(0, NCH // 2, pair, 0)
            pltpu.make_async_copy(h2_hbm.at[idx0], rows0, sem0).wait()
            pltpu.sync_copy(acc.at[pl.ds(0, NHALF), :],
                            out_hbm.at[half, :, q, col, :])

    return sb2(src, dst, p2, h2v)


def _t3_body(acc_ref, part_ref, tabS_ref, tabD_ref, wsum_ref, cnt_ref,
             c2_ref, h2_ref, b2_ref, out_ref):
    accsum = jnp.sum(acc_ref[...], axis=1)
    ssum = part_ref[...]
    as2 = tabS_ref[...][:, 0:1]
    ad2 = tabD_ref[...][:, 0:1]
    la = wsum_ref[...] / jnp.maximum(cnt_ref[...], 1.0)
    c2 = c2_ref[...][:, 0:1]
    z = as2 + ad2 + la * c2
    z = jnp.maximum(z, 0.2 * z)
    ps = jnp.exp(z)
    h2b = h2_ref[...]
    out = (accsum + ps * h2b) / (ssum + ps + 1e-16)
    out_ref[...] = out + b2_ref[...]


def _t3(acc2, part2, tab2S, tab2D, wsum, cnt, c2arr, h2, b2):
    return pl.pallas_call(
        _t3_body,
        grid=(N // BN,),
        in_specs=[
            pl.BlockSpec((BN, 4, DOUT), lambda i: (i, 0, 0)),
            pl.BlockSpec((BN, 1), lambda i: (i, 0)),
            pl.BlockSpec((BN, 16), lambda i: (i, 0)),
            pl.BlockSpec((BN, 16), lambda i: (i, 0)),
            pl.BlockSpec((BN, 1), lambda i: (i, 0)),
            pl.BlockSpec((BN, 1), lambda i: (i, 0)),
            pl.BlockSpec((1, 16), lambda i: (0, 0)),
            pl.BlockSpec((BN, DOUT), lambda i: (i, 0)),
            pl.BlockSpec((1, DOUT), lambda i: (0, 0)),
        ],
        out_specs=pl.BlockSpec((BN, DOUT), lambda i: (i, 0)),
        out_shape=jax.ShapeDtypeStruct((N, DOUT), jnp.float32),
    )(acc2, part2, tab2S, tab2D, wsum, cnt, c2arr, h2, b2)


def kernel(x, edge_index, edge_weight, W1, a_src1, a_dst1, We1, ae1, b1,
           W2, a_src2, a_dst2, We2, ae2, b2):
    src = edge_index[0]
    dst = edge_index[1]
    pad = jnp.zeros((KB,), dtype=edge_index.dtype)
    srcp = jnp.concatenate([src, pad])
    dstp = jnp.concatenate([dst, pad])
    h1, tabS, tabD, c1arr = _t1(x, W1, a_src1.reshape(1, F1),
                                a_dst1.reshape(1, F1), We1, ae1.reshape(1, F1))
    p1, part1 = _sa1(src, dst, edge_weight, tabS, tabD, c1arr.reshape(16))
    red1 = _reduce_partials(part1.reshape(NW * 32 * SPAN1), 32 * SPAN1, SPAN1)
    psum = red1[:N * 8].reshape(N, 8)
    wc = red1[N * 8:N * 10].reshape(N, 2)
    wsum = wc[:, 0:1]
    cnt = wc[:, 1:2]
    acc1 = _sb1(srcp, dstp, p1, h1.reshape(N * NW, 16))
    h2, tab2S, tab2D, c2arr = _t2(
        acc1.reshape(N, F1), psum, wsum, cnt, tabS, tabD, c1arr, h1,
        b1.reshape(1, F1), W2, a_src2.reshape(1, DOUT), a_dst2.reshape(1, DOUT),
        We2, ae2.reshape(1, DOUT))
    p2, part2 = _sa2(src, dst, edge_weight, tab2S, tab2D, c2arr.reshape(16))
    red2 = _reduce_partials(part2.reshape(NW * M2), M2, SPAN2)
    ssum2 = red2[:N].reshape(N, 1)
    acc2 = _sb2(srcp, dstp, p2, h2.reshape(N * 8, 16))
    out = _t3(acc2.reshape(N, 4, DOUT), ssum2, tab2S, tab2D, wsum, cnt,
              c2arr, h2, b2.reshape(1, DOUT))
    return out


# speedup vs baseline: 1.3989x; 1.3989x over previous
import functools
import jax
import jax.numpy as jnp
from jax import lax
from jax.experimental import pallas as pl
from jax.experimental.pallas import tpu as pltpu
from jax.experimental.pallas import tpu_sc as plsc

N = 10000
E = 320000
DIN = 128
HID = 64
H = 8
F1 = 512
DOUT = 128

NC = 2
NS = 16
NW = NC * NS

NHALF = N // 2
NACC = NHALF + 8
EC = E // NW
KA = 400
KB = 800
BN = 1000

NP16 = 640
M1 = 100000
SPAN1 = 3136
M2 = NP16 * 16
SPAN2 = M2 // NW


def _iota16():
    return lax.broadcasted_iota(jnp.int32, (16,), 0)


def _t1_body(x_ref, w_ref, asrc_ref, adst_ref, we_ref, ae_ref,
             h_ref, tabS_ref, tabD_ref, c_ref):
    h = jnp.dot(x_ref[...], w_ref[...], preferred_element_type=jnp.float32)
    h_ref[...] = h
    row = lax.broadcasted_iota(jnp.int32, (F1, 16), 0)
    col = lax.broadcasted_iota(jnp.int32, (F1, 16), 1)
    blk = (row // HID) == (col % H)
    blk8 = blk[:, 0:8]
    As = jnp.where(blk8, asrc_ref[...].T, 0.0)
    Ad = jnp.where(blk8, adst_ref[...].T, 0.0)
    as1 = jnp.dot(h, As, preferred_element_type=jnp.float32)
    ad1 = jnp.dot(h, Ad, preferred_element_type=jnp.float32)
    tabS_ref[...] = jnp.concatenate([as1, as1], axis=1)
    tabD_ref[...] = jnp.concatenate([ad1, ad1], axis=1)
    pe = we_ref[...] * ae_ref[...]
    M16 = jnp.where(blk, 1.0, 0.0)
    c_ref[...] = jnp.dot(pe, M16, preferred_element_type=jnp.float32)


def _t1(x, W1, a_src1, a_dst1, We1, ae1):
    return pl.pallas_call(
        _t1_body,
        grid=(N // BN,),
        in_specs=[
            pl.BlockSpec((BN, DIN), lambda i: (i, 0)),
            pl.BlockSpec((DIN, F1), lambda i: (0, 0)),
            pl.BlockSpec((1, F1), lambda i: (0, 0)),
            pl.BlockSpec((1, F1), lambda i: (0, 0)),
            pl.BlockSpec((1, F1), lambda i: (0, 0)),
            pl.BlockSpec((1, F1), lambda i: (0, 0)),
        ],
        out_specs=[
            pl.BlockSpec((BN, F1), lambda i: (i, 0)),
            pl.BlockSpec((BN, 16), lambda i: (i, 0)),
            pl.BlockSpec((BN, 16), lambda i: (i, 0)),
            pl.BlockSpec((1, 16), lambda i: (0, 0)),
        ],
        out_shape=[
            jax.ShapeDtypeStruct((N, F1), jnp.float32),
            jax.ShapeDtypeStruct((N, 16), jnp.float32),
            jax.ShapeDtypeStruct((N, 16), jnp.float32),
            jax.ShapeDtypeStruct((1, 16), jnp.float32),
        ],
    )(x, W1, a_src1, a_dst1, We1, ae1)


def _sa1(src, dst, w, tabS, tabD, c1arr):
    mesh = plsc.VectorSubcoreMesh(core_axis_name="c", subcore_axis_name="s")

    @functools.partial(
        pl.kernel,
        out_type=[
            jax.ShapeDtypeStruct((H * E + KB,), jnp.float32),
            jax.ShapeDtypeStruct((NW, 32 * SPAN1), jnp.float32),
        ],
        mesh=mesh,
        compiler_params=pltpu.CompilerParams(use_tc_tiling_on_sc=False, needs_layout_passes=False),
        scratch_types=[
            pltpu.VMEM((KA,), jnp.int32),
            pltpu.VMEM((KA,), jnp.int32),
            pltpu.VMEM((KA,), jnp.float32),
            pltpu.VMEM((KA, 16), jnp.float32),
            pltpu.VMEM((KA, 16), jnp.float32),
            pltpu.VMEM((H * KA,), jnp.float32),
            pltpu.VMEM((N * 8,), jnp.float32),
            pltpu.VMEM((N * 2,), jnp.float32),
            pltpu.VMEM((16,), jnp.float32),
            pltpu.SemaphoreType.DMA,
            pltpu.SemaphoreType.DMA,
        ],
    )
    def sa1(src_hbm, dst_hbm, w_hbm, tabS_hbm, tabD_hbm, c1_hbm,
            p_hbm, part_hbm,
            src_v, dst_v, w_v, rowsS, rowsD, pbuf, accP, accWC,
            c1_v, sem1, sem2):
        wid = lax.axis_index("s") * NC + lax.axis_index("c")
        iota = _iota16()
        zero16 = jnp.zeros((16,), jnp.float32)
        lane_lt8 = iota < 8
        lane_lt2 = iota < 2
        iota_ka = iota * KA

        pltpu.sync_copy(c1_hbm, c1_v)
        c1v = c1_v[...]

        def zP(i, _):
            accP[pl.ds(i * 16, 16)] = zero16
            return 0
        lax.fori_loop(0, (N * 8) // 16, zP, 0)

        def zWC(i, _):
            accWC[pl.ds(i * 16, 16)] = zero16
            return 0
        lax.fori_loop(0, (N * 2) // 16, zWC, 0)

        base0 = wid * EC

        def chunk_body(ci, _):
            base = base0 + ci * KA
            pltpu.sync_copy(src_hbm.at[pl.ds(base, KA)], src_v)
            pltpu.sync_copy(dst_hbm.at[pl.ds(base, KA)], dst_v)
            pltpu.sync_copy(w_hbm.at[pl.ds(base, KA)], w_v)
            d1 = pltpu.async_copy(tabS_hbm.at[src_v], rowsS, sem1)
            d2 = pltpu.async_copy(tabD_hbm.at[dst_v], rowsD, sem2)
            d1.wait()
            d2.wait()

            def edge_grp(j, _):
                jf = jnp.full((16,), j, jnp.int32)
                dv16 = plsc.load_gather(dst_v, [jf])
                wsp = plsc.load_gather(w_v, [jf])
                vas = plsc.load_gather(rowsS, [jf, iota])
                vad = plsc.load_gather(rowsD, [jf, iota])
                z = vas + vad + wsp * c1v
                z = jnp.maximum(z, 0.2 * z)
                p16 = jnp.exp(z)
                plsc.store_scatter(pbuf, [iota_ka + j], p16, mask=lane_lt8)
                plsc.addupdate_scatter(accP, [dv16 * 8 + iota], p16,
                                       mask=lane_lt8)
                plsc.addupdate_scatter(accWC, [dv16 * 2 + iota],
                                       jnp.where(iota == 0, wsp, 1.0),
                                       mask=lane_lt2)
                return 0
            lax.fori_loop(0, KA, edge_grp, 0)
            for h in range(H):
                pltpu.sync_copy(pbuf.at[pl.ds(h * KA, KA)],
                                p_hbm.at[pl.ds(h * E + base, KA)])
            return 0
        lax.fori_loop(0, EC // KA, chunk_body, 0)
        pltpu.sync_copy(accP, part_hbm.at[wid, pl.ds(0, N * 8)])
        pltpu.sync_copy(accWC, part_hbm.at[wid, pl.ds(N * 8, N * 2)])

    return sa1(src, dst, w, tabS, tabD, c1arr)


def _reduce_partials(part, mprow, span):
    mesh = plsc.VectorSubcoreMesh(core_axis_name="c", subcore_axis_name="s")

    @functools.partial(
        pl.kernel,
        out_type=jax.ShapeDtypeStruct((NW * span,), jnp.float32),
        mesh=mesh,
        compiler_params=pltpu.CompilerParams(use_tc_tiling_on_sc=False, needs_layout_passes=False),
        scratch_types=[
            pltpu.VMEM((span,), jnp.float32),
            pltpu.VMEM((span,), jnp.float32),
        ],
    )
    def red_k(part_hbm, out_hbm, buf, accv):
        wid = lax.axis_index("s") * NC + lax.axis_index("c")
        off = wid * span
        zero16 = jnp.zeros((16,), jnp.float32)

        def zbody(i, _):
            accv[pl.ds(i * 16, 16)] = zero16
            return 0
        lax.fori_loop(0, span // 16, zbody, 0)

        def tbody(t, _):
            pltpu.sync_copy(part_hbm.at[pl.ds(t * mprow + off, span)], buf)

            def abody(i, _):
                sl = pl.ds(i * 16, 16)
                accv[sl] = accv[sl] + buf[sl]
                return 0
            lax.fori_loop(0, span // 16, abody, 0)
            return 0
        lax.fori_loop(0, NW, tbody, 0)
        pltpu.sync_copy(accv, out_hbm.at[pl.ds(off, span)])

    return red_k(part)


def _sb1(src, dst, p1, h1v):
    mesh = plsc.VectorSubcoreMesh(core_axis_name="c", subcore_axis_name="s")

    @functools.partial(
        pl.kernel,
        out_type=jax.ShapeDtypeStruct((2, NHALF, NW, 16), jnp.float32),
        mesh=mesh,
        compiler_params=pltpu.CompilerParams(use_tc_tiling_on_sc=False, needs_layout_passes=False),
        scratch_types=[
            pltpu.VMEM((KB,), jnp.int32),
            pltpu.VMEM((KB,), jnp.int32),
            pltpu.VMEM((KB,), jnp.float32),
            pltpu.VMEM((KB,), jnp.float32),
            pltpu.VMEM((KB,), jnp.int32),
            pltpu.VMEM((KB,), jnp.int32),
            pltpu.VMEM((KB,), jnp.int32),
            pltpu.VMEM((KB,), jnp.int32),
            pltpu.VMEM((KB, 16), jnp.float32),
            pltpu.VMEM((KB, 16), jnp.float32),
            pltpu.VMEM((NACC, 16), jnp.float32),
            pltpu.SemaphoreType.DMA,
            pltpu.SemaphoreType.DMA,
            pltpu.SemaphoreType.DMA,
            pltpu.SemaphoreType.DMA,
            pltpu.SemaphoreType.DMA,
        ],
    )
    def sb1(src_hbm, dst_hbm, p_hbm, h1_hbm, out_hbm,
            src_v, dst_v, p0, p1_, idx0, idx1, dl0, dl1, rows0, rows1,
            acc, sem0, sem1, semA, semB, semC):
        wid = lax.axis_index("s") * NC + lax.axis_index("c")
        head = wid // 4
        iota = _iota16()
        zero16 = jnp.zeros((16,), jnp.float32)
        NCH = E // KB

        for half in range(2):
            def zbody(i, _):
                acc[i] = zero16
                return 0
            lax.fori_loop(0, NACC, zbody, 0)

            def load_prep(ci, p_b, idx_b, dl_b):
                base = ci * KB
                da = pltpu.async_copy(src_hbm.at[pl.ds(base, KB)], src_v, semA)
                db = pltpu.async_copy(dst_hbm.at[pl.ds(base, KB)], dst_v, semB)
                dc = pltpu.async_copy(p_hbm.at[pl.ds(head * E + base, KB)],
                                      p_b, semC)
                da.wait()
                db.wait()
                dc.wait()

                def prep(g, _):
                    sl = pl.ds(g * 16, 16)
                    sv = src_v[sl]
                    idx_b[sl] = sv * NW + wid
                    dv = dst_v[sl] - half * NHALF
                    m = (dv >= 0) & (dv < NHALF)
                    dl_b[sl] = jnp.where(m, dv, NHALF)
                    return 0
                lax.fori_loop(0, KB // 16, prep, 0)

            def compute(rows_b, p_b, dl_b):
                def edge_grp(g, _):
                    sl = pl.ds(g * 16, 16)
                    pvec = p_b[sl]
                    dlvec = dl_b[sl]
                    for u in range(16):
                        j = g * 16 + u
                        plsc.addupdate(acc.at[dlvec[u]],
                                       rows_b[j] * pvec[u])
                    return 0
                lax.fori_loop(0, KB // 16, edge_grp, 0)

            load_prep(0, p0, idx0, dl0)
            pltpu.async_copy(h1_hbm.at[idx0], rows0, sem0)

            def pair(k, _):
                c0 = 2 * k
                load_prep(c0 + 1, p1_, idx1, dl1)
                pltpu.async_copy(h1_hbm.at[idx1], rows1, sem1)
                pltpu.make_async_copy(h1_hbm.at[idx0], rows0, sem0).wait()
                compute(rows0, p0, dl0)
                load_prep(c0 + 2, p0, idx0, dl0)
                pltpu.async_copy(h1_hbm.at[idx0], rows0, sem0)
                pltpu.make_async_copy(h1_hbm.at[idx1], rows1, sem1).wait()
                compute(rows1, p1_, dl1)
                return 0
            lax.fori_loop(0, NCH // 2, pair, 0)
            pltpu.make_async_copy(h1_hbm.at[idx0], rows0, sem0).wait()
            pltpu.sync_copy(acc.at[pl.ds(0, NHALF), :],
                            out_hbm.at[half, :, wid, :])

    return sb1(src, dst, p1, h1v)


def _t2_body(acc_ref, psum_ref, wsum_ref, cnt_ref, tabS_ref, tabD_ref,
             c_ref, h_ref, b_ref, w2_ref,
             asrc2_ref, adst2_ref, we2_ref, ae2_ref,
             h2_ref, tab2S_ref, tab2D_ref, c2_ref):
    psum = psum_ref[...]
    la = wsum_ref[...] / jnp.maximum(cnt_ref[...], 1.0)
    as1 = tabS_ref[...][:, 0:8]
    ad1 = tabD_ref[...][:, 0:8]
    c1 = c_ref[...][:, 0:8]
    z = as1 + ad1 + la * c1
    z = jnp.maximum(z, 0.2 * z)
    pself = jnp.exp(z)
    ssum = psum + pself + 1e-16
    h1b = h_ref[...]
    accb = acc_ref[...]
    rrow = lax.broadcasted_iota(jnp.int32, (H, F1), 0)
    rcol = lax.broadcasted_iota(jnp.int32, (H, F1), 1)
    R = jnp.where(rrow == (rcol // HID), 1.0, 0.0)
    pw = jnp.dot(pself, R, preferred_element_type=jnp.float32)
    sw = jnp.dot(ssum, R, preferred_element_type=jnp.float32)
    h1o = (accb + pw * h1b) / sw + b_ref[...]
    act = jnp.where(h1o > 0, h1o, jnp.exp(jnp.minimum(h1o, 0.0)) - 1.0)
    h2 = jnp.dot(act, w2_ref[...], preferred_element_type=jnp.float32)
    h2_ref[...] = h2
    as2 = jnp.sum(h2 * asrc2_ref[...], axis=1, keepdims=True)
    ad2 = jnp.sum(h2 * adst2_ref[...], axis=1, keepdims=True)
    ones16 = jnp.ones((1, 16), jnp.float32)
    tab2S_ref[...] = as2 * ones16
    tab2D_ref[...] = ad2 * ones16
    c2 = jnp.sum(we2_ref[...] * ae2_ref[...], axis=1, keepdims=True)
    c2_ref[...] = c2 * ones16


def _t2(acc1, psum, wsum, cnt, tabS, tabD, c1arr, h1, b1, W2,
        a_src2, a_dst2, We2, ae2):
    return pl.pallas_call(
        _t2_body,
        grid=(N // BN,),
        in_specs=[
            pl.BlockSpec((BN, F1), lambda i: (i, 0)),
            pl.BlockSpec((BN, 8), lambda i: (i, 0)),
            pl.BlockSpec((BN, 1), lambda i: (i, 0)),
            pl.BlockSpec((BN, 1), lambda i: (i, 0)),
            pl.BlockSpec((BN, 16), lambda i: (i, 0)),
            pl.BlockSpec((BN, 16), lambda i: (i, 0)),
            pl.BlockSpec((1, 16), lambda i: (0, 0)),
            pl.BlockSpec((BN, F1), lambda i: (i, 0)),
            pl.BlockSpec((1, F1), lambda i: (0, 0)),
            pl.BlockSpec((F1, DOUT), lambda i: (0, 0)),
            pl.BlockSpec((1, DOUT), lambda i: (0, 0)),
            pl.BlockSpec((1, DOUT), lambda i: (0, 0)),
            pl.BlockSpec((1, DOUT), lambda i: (0, 0)),
            pl.BlockSpec((1, DOUT), lambda i: (0, 0)),
        ],
        out_specs=[
            pl.BlockSpec((BN, DOUT), lambda i: (i, 0)),
            pl.BlockSpec((BN, 16), lambda i: (i, 0)),
            pl.BlockSpec((BN, 16), lambda i: (i, 0)),
            pl.BlockSpec((1, 16), lambda i: (0, 0)),
        ],
        out_shape=[
            jax.ShapeDtypeStruct((N, DOUT), jnp.float32),
            jax.ShapeDtypeStruct((N, 16), jnp.float32),
            jax.ShapeDtypeStruct((N, 16), jnp.float32),
            jax.ShapeDtypeStruct((1, 16), jnp.float32),
        ],
    )(acc1, psum, wsum, cnt, tabS, tabD, c1arr, h1, b1, W2,
      a_src2, a_dst2, We2, ae2)


def _sa2(src, dst, w, tab2S, tab2D, c2arr):
    mesh = plsc.VectorSubcoreMesh(core_axis_name="c", subcore_axis_name="s")

    @functools.partial(
        pl.kernel,
        out_type=[
            jax.ShapeDtypeStruct((E + KB,), jnp.float32),
            jax.ShapeDtypeStruct((NW, NP16, 16), jnp.float32),
        ],
        mesh=mesh,
        compiler_params=pltpu.CompilerParams(use_tc_tiling_on_sc=False, needs_layout_passes=False),
        scratch_types=[
            pltpu.VMEM((KA,), jnp.int32),
            pltpu.VMEM((KA,), jnp.int32),
            pltpu.VMEM((KA,), jnp.float32),
            pltpu.VMEM((KA, 16), jnp.float32),
            pltpu.VMEM((KA, 16), jnp.float32),
            pltpu.VMEM((KA,), jnp.float32),
            pltpu.VMEM((NP16, 16), jnp.float32),
            pltpu.VMEM((16,), jnp.float32),
            pltpu.SemaphoreType.DMA,
            pltpu.SemaphoreType.DMA,
        ],
    )
    def sa2(src_hbm, dst_hbm, w_hbm, tabS_hbm, tabD_hbm, c2_hbm,
            p_hbm, part_hbm,
            src_v, dst_v, w_v, rowsS, rowsD, p_v, accS, c2_v,
            sem1, sem2):
        wid = lax.axis_index("s") * NC + lax.axis_index("c")
        iota = _iota16()
        zero16 = jnp.zeros((16,), jnp.float32)
        izero = jnp.zeros((16,), jnp.int32)
        lane0 = iota == 0

        pltpu.sync_copy(c2_hbm, c2_v)
        c2v = c2_v[...]

        def zS(i, _):
            accS[i] = zero16
            return 0
        lax.fori_loop(0, NP16, zS, 0)

        base0 = wid * EC

        def chunk_body(ci, _):
            base = base0 + ci * KA
            pltpu.sync_copy(src_hbm.at[pl.ds(base, KA)], src_v)
            pltpu.sync_copy(dst_hbm.at[pl.ds(base, KA)], dst_v)
            pltpu.sync_copy(w_hbm.at[pl.ds(base, KA)], w_v)
            d1 = pltpu.async_copy(tabS_hbm.at[src_v], rowsS, sem1)
            d2 = pltpu.async_copy(tabD_hbm.at[dst_v], rowsD, sem2)
            d1.wait()
            d2.wait()

            def edge_grp(g, _):
                sl = pl.ds(g * 16, 16)
                dvec = dst_v[sl]
                wvec = w_v[sl]
                for u in range(16):
                    j = g * 16 + u
                    dv = dvec[u]
                    z = rowsS[j] + rowsD[j] + wvec[u] * c2v
                    z = jnp.maximum(z, 0.2 * z)
                    p16 = jnp.exp(z)
                    plsc.store_scatter(p_v, [izero + j], p16, mask=lane0)
                    plsc.addupdate(accS.at[dv // 16],
                                   jnp.where(iota == (dv % 16), p16, 0.0))
                return 0
            lax.fori_loop(0, KA // 16, edge_grp, 0)
            pltpu.sync_copy(p_v, p_hbm.at[pl.ds(base, KA)])
            return 0
        lax.fori_loop(0, EC // KA, chunk_body, 0)
        pltpu.sync_copy(accS, part_hbm.at[wid])

    return sa2(src, dst, w, tab2S, tab2D, c2arr)


def _sb2(src, dst, p2, h2v):
    mesh = plsc.VectorSubcoreMesh(core_axis_name="c", subcore_axis_name="s")
    EQ = E // 4

    @functools.partial(
        pl.kernel,
        out_type=jax.ShapeDtypeStruct((2, NHALF, 4, 8, 16), jnp.float32),
        mesh=mesh,
        compiler_params=pltpu.CompilerParams(use_tc_tiling_on_sc=False, needs_layout_passes=False),
        scratch_types=[
            pltpu.VMEM((KB,), jnp.int32),
            pltpu.VMEM((KB,), jnp.int32),
            pltpu.VMEM((KB,), jnp.float32),
            pltpu.VMEM((KB,), jnp.float32),
            pltpu.VMEM((KB,), jnp.int32),
            pltpu.VMEM((KB,), jnp.int32),
            pltpu.VMEM((KB,), jnp.int32),
            pltpu.VMEM((KB,), jnp.int32),
            pltpu.VMEM((KB, 16), jnp.float32),
            pltpu.VMEM((KB, 16), jnp.float32),
            pltpu.VMEM((NACC, 16), jnp.float32),
            pltpu.SemaphoreType.DMA,
            pltpu.SemaphoreType.DMA,
            pltpu.SemaphoreType.DMA,
            pltpu.SemaphoreType.DMA,
            pltpu.SemaphoreType.DMA,
        ],
    )
    def sb2(src_hbm, dst_hbm, p_hbm, h2_hbm, out_hbm,
            src_v, dst_v, p0, p1_, idx0, idx1, dl0, dl1, rows0, rows1,
            acc, sem0, sem1, semA, semB, semC):
        wid = lax.axis_index("s") * NC + lax.axis_index("c")
        q = wid // 8
        col = wid % 8
        iota = _iota16()
        zero16 = jnp.zeros((16,), jnp.float32)
        ebase = q * EQ
        NCH = EQ // KB

        for half in range(2):
            def zbody(i, _):
                acc[i] = zero16
                return 0
            lax.fori_loop(0, NACC, zbody, 0)

            def load_prep(ci, p_b, idx_b, dl_b):
                base = ebase + ci * KB
                da = pltpu.async_copy(src_hbm.at[pl.ds(base, KB)], src_v, semA)
                db = pltpu.async_copy(dst_hbm.at[pl.ds(base, KB)], dst_v, semB)
                dc = pltpu.async_copy(p_hbm.at[pl.ds(base, KB)], p_b, semC)
                da.wait()
                db.wait()
                dc.wait()

                def prep(g, _):
                    sl = pl.ds(g * 16, 16)
                    sv = src_v[sl]
                    idx_b[sl] = sv * 8 + col
                    dv = dst_v[sl] - half * NHALF
                    m = (dv >= 0) & (dv < NHALF)
                    dl_b[sl] = jnp.where(m, dv, NHALF)
                    return 0
                lax.fori_loop(0, KB // 16, prep, 0)

            def compute(rows_b, p_b, dl_b):
                def edge_grp(g, _):
                    sl = pl.ds(g * 16, 16)
                    pvec = p_b[sl]
                    dlvec = dl_b[sl]
                    for u in range(16):
                        j = g * 16 + u
                        plsc.addupdate(acc.at[dlvec[u]],
                                       rows_b[j] * pvec[u])
                    return 0
                lax.fori_loop(0, KB // 16, edge_grp, 0)

            load_prep(0, p0, idx0, dl0)
            pltpu.async_copy(h2_hbm.at[idx0], rows0, sem0)

            def pair(k, _):
                c0 = 2 * k
                load_prep(c0 + 1, p1_, idx1, dl1)
                pltpu.async_copy(h2_hbm.at[idx1], rows1, sem1)
                pltpu.make_async_copy(h2_hbm.at[idx0], rows0, sem0).wait()
                compute(rows0, p0, dl0)
                load_prep(c0 + 2, p0, idx0, dl0)
                pltpu.async_copy(h2_hbm.at[idx0], rows0, sem0)
                pltpu.make_async_copy(h2_hbm.at[idx1], rows1, sem1).wait()
                compute(rows1, p1_, dl1)
                return 0
            lax.fori_loop(0, NCH // 2, pair, 0)
            pltpu.make_async_copy(h2_hbm.at[idx0], rows0, sem0).wait()
            pltpu.sync_copy(acc.at[pl.ds(0, NHALF), :],
                            out_hbm.at[half, :, q, col, :])

    return sb2(src, dst, p2, h2v)


def _t3_body(acc_ref, part_ref, tabS_ref, tabD_ref, wsum_ref, cnt_ref,
             c2_ref, h2_ref, b2_ref, out_ref):
    accsum = jnp.sum(acc_ref[...], axis=1)
    ssum = part_ref[...]
    as2 = tabS_ref[...][:, 0:1]
    ad2 = tabD_ref[...][:, 0:1]
    la = wsum_ref[...] / jnp.maximum(cnt_ref[...], 1.0)
    c2 = c2_ref[...][:, 0:1]
    z = as2 + ad2 + la * c2
    z = jnp.maximum(z, 0.2 * z)
    ps = jnp.exp(z)
    h2b = h2_ref[...]
    out = (accsum + ps * h2b) / (ssum + ps + 1e-16)
    out_ref[...] = out + b2_ref[...]


def _t3(acc2, part2, tab2S, tab2D, wsum, cnt, c2arr, h2, b2):
    return pl.pallas_call(
        _t3_body,
        grid=(N // BN,),
        in_specs=[
            pl.BlockSpec((BN, 4, DOUT), lambda i: (i, 0, 0)),
            pl.BlockSpec((BN, 1), lambda i: (i, 0)),
            pl.BlockSpec((BN, 16), lambda i: (i, 0)),
            pl.BlockSpec((BN, 16), lambda i: (i, 0)),
            pl.BlockSpec((BN, 1), lambda i: (i, 0)),
            pl.BlockSpec((BN, 1), lambda i: (i, 0)),
            pl.BlockSpec((1, 16), lambda i: (0, 0)),
            pl.BlockSpec((BN, DOUT), lambda i: (i, 0)),
            pl.BlockSpec((1, DOUT), lambda i: (0, 0)),
        ],
        out_specs=pl.BlockSpec((BN, DOUT), lambda i: (i, 0)),
        out_shape=jax.ShapeDtypeStruct((N, DOUT), jnp.float32),
    )(acc2, part2, tab2S, tab2D, wsum, cnt, c2arr, h2, b2)


def kernel(x, edge_index, edge_weight, W1, a_src1, a_dst1, We1, ae1, b1,
           W2, a_src2, a_dst2, We2, ae2, b2):
    src = edge_index[0]
    dst = edge_index[1]
    pad = jnp.zeros((KB,), dtype=edge_index.dtype)
    srcp = jnp.concatenate([src, pad])
    dstp = jnp.concatenate([dst, pad])
    h1, tabS, tabD, c1arr = _t1(x, W1, a_src1.reshape(1, F1),
                                a_dst1.reshape(1, F1), We1, ae1.reshape(1, F1))
    p1, part1 = _sa1(src, dst, edge_weight, tabS, tabD, c1arr.reshape(16))
    red1 = _reduce_partials(part1.reshape(NW * 32 * SPAN1), 32 * SPAN1, SPAN1)
    psum = red1[:N * 8].reshape(N, 8)
    wc = red1[N * 8:N * 10].reshape(N, 2)
    wsum = wc[:, 0:1]
    cnt = wc[:, 1:2]
    acc1 = _sb1(srcp, dstp, p1, h1.reshape(N * NW, 16))
    h2, tab2S, tab2D, c2arr = _t2(
        acc1.reshape(N, F1), psum, wsum, cnt, tabS, tabD, c1arr, h1,
        b1.reshape(1, F1), W2, a_src2.reshape(1, DOUT), a_dst2.reshape(1, DOUT),
        We2, ae2.reshape(1, DOUT))
    p2, part2 = _sa2(src, dst, edge_weight, tab2S, tab2D, c2arr.reshape(16))
    red2 = _reduce_partials(part2.reshape(NW * M2), M2, SPAN2)
    ssum2 = red2[:N].reshape(N, 1)
    acc2 = _sb2(srcp, dstp, p2, h2.reshape(N * 8, 16))
    out = _t3(acc2.reshape(N, 4, DOUT), ssum2, tab2S, tab2D, wsum, cnt,
              c2arr, h2, b2.reshape(1, DOUT))
    return out


# speedup vs baseline: 1.4805x; 1.0584x over previous
import functools
import jax
import jax.numpy as jnp
from jax import lax
from jax.experimental import pallas as pl
from jax.experimental.pallas import tpu as pltpu
from jax.experimental.pallas import tpu_sc as plsc

N = 10000
E = 320000
DIN = 128
HID = 64
H = 8
F1 = 512
DOUT = 128

NC = 2
NS = 16
NW = NC * NS

NHALF = N // 2
NACC = NHALF + 8
EC = E // NW
KA = 400
KB = 800
BN = 1000

NP16 = 640
M1 = 100000
SPAN1 = 3136
M2 = NP16 * 16
SPAN2 = M2 // NW


def _iota16():
    return lax.broadcasted_iota(jnp.int32, (16,), 0)


def _t1_body(x_ref, w_ref, asrc_ref, adst_ref, we_ref, ae_ref,
             h_ref, tabS_ref, tabD_ref, c_ref):
    h = jnp.dot(x_ref[...], w_ref[...], preferred_element_type=jnp.float32)
    h_ref[...] = h
    row = lax.broadcasted_iota(jnp.int32, (F1, 16), 0)
    col = lax.broadcasted_iota(jnp.int32, (F1, 16), 1)
    blk = (row // HID) == (col % H)
    blk8 = blk[:, 0:8]
    As = jnp.where(blk8, asrc_ref[...].T, 0.0)
    Ad = jnp.where(blk8, adst_ref[...].T, 0.0)
    as1 = jnp.dot(h, As, preferred_element_type=jnp.float32)
    ad1 = jnp.dot(h, Ad, preferred_element_type=jnp.float32)
    tabS_ref[...] = jnp.concatenate([as1, as1], axis=1)
    tabD_ref[...] = jnp.concatenate([ad1, ad1], axis=1)
    pe = we_ref[...] * ae_ref[...]
    M16 = jnp.where(blk, 1.0, 0.0)
    c_ref[...] = jnp.dot(pe, M16, preferred_element_type=jnp.float32)


def _t1(x, W1, a_src1, a_dst1, We1, ae1):
    return pl.pallas_call(
        _t1_body,
        grid=(N // BN,),
        in_specs=[
            pl.BlockSpec((BN, DIN), lambda i: (i, 0)),
            pl.BlockSpec((DIN, F1), lambda i: (0, 0)),
            pl.BlockSpec((1, F1), lambda i: (0, 0)),
            pl.BlockSpec((1, F1), lambda i: (0, 0)),
            pl.BlockSpec((1, F1), lambda i: (0, 0)),
            pl.BlockSpec((1, F1), lambda i: (0, 0)),
        ],
        out_specs=[
            pl.BlockSpec((BN, F1), lambda i: (i, 0)),
            pl.BlockSpec((BN, 16), lambda i: (i, 0)),
            pl.BlockSpec((BN, 16), lambda i: (i, 0)),
            pl.BlockSpec((1, 16), lambda i: (0, 0)),
        ],
        out_shape=[
            jax.ShapeDtypeStruct((N, F1), jnp.float32),
            jax.ShapeDtypeStruct((N, 16), jnp.float32),
            jax.ShapeDtypeStruct((N, 16), jnp.float32),
            jax.ShapeDtypeStruct((1, 16), jnp.float32),
        ],
    )(x, W1, a_src1, a_dst1, We1, ae1)


def _sa1(src, dst, w, tabS, tabD, c1arr):
    mesh = plsc.VectorSubcoreMesh(core_axis_name="c", subcore_axis_name="s")

    @functools.partial(
        pl.kernel,
        out_type=[
            jax.ShapeDtypeStruct((H * E + KB,), jnp.float32),
            jax.ShapeDtypeStruct((NW, 32 * SPAN1), jnp.float32),
        ],
        mesh=mesh,
        compiler_params=pltpu.CompilerParams(use_tc_tiling_on_sc=False, needs_layout_passes=False),
        scratch_types=[
            pltpu.VMEM((KA,), jnp.int32),
            pltpu.VMEM((KA,), jnp.int32),
            pltpu.VMEM((KA,), jnp.float32),
            pltpu.VMEM((KA, 16), jnp.float32),
            pltpu.VMEM((KA, 16), jnp.float32),
            pltpu.VMEM((H * KA,), jnp.float32),
            pltpu.VMEM((N * 8,), jnp.float32),
            pltpu.VMEM((N * 2,), jnp.float32),
            pltpu.VMEM((16,), jnp.float32),
            pltpu.SemaphoreType.DMA,
            pltpu.SemaphoreType.DMA,
        ],
    )
    def sa1(src_hbm, dst_hbm, w_hbm, tabS_hbm, tabD_hbm, c1_hbm,
            p_hbm, part_hbm,
            src_v, dst_v, w_v, rowsS, rowsD, pbuf, accP, accWC,
            c1_v, sem1, sem2):
        wid = lax.axis_index("s") * NC + lax.axis_index("c")
        iota = _iota16()
        zero16 = jnp.zeros((16,), jnp.float32)
        lane_lt8 = iota < 8
        lane_lt2 = iota < 2
        iota_ka = iota * KA

        pltpu.sync_copy(c1_hbm, c1_v)
        c1v = c1_v[...]

        def zP(i, _):
            accP[pl.ds(i * 16, 16)] = zero16
            return 0
        lax.fori_loop(0, (N * 8) // 16, zP, 0)

        def zWC(i, _):
            accWC[pl.ds(i * 16, 16)] = zero16
            return 0
        lax.fori_loop(0, (N * 2) // 16, zWC, 0)

        base0 = wid * EC

        def chunk_body(ci, _):
            base = base0 + ci * KA
            pltpu.sync_copy(src_hbm.at[pl.ds(base, KA)], src_v)
            pltpu.sync_copy(dst_hbm.at[pl.ds(base, KA)], dst_v)
            pltpu.sync_copy(w_hbm.at[pl.ds(base, KA)], w_v)
            d1 = pltpu.async_copy(tabS_hbm.at[src_v], rowsS, sem1)
            d2 = pltpu.async_copy(tabD_hbm.at[dst_v], rowsD, sem2)
            d1.wait()
            d2.wait()

            def edge_grp(j, _):
                jf = jnp.full((16,), j, jnp.int32)
                dv16 = plsc.load_gather(dst_v, [jf])
                wsp = plsc.load_gather(w_v, [jf])
                vas = plsc.load_gather(rowsS, [jf, iota])
                vad = plsc.load_gather(rowsD, [jf, iota])
                z = vas + vad + wsp * c1v
                z = jnp.maximum(z, 0.2 * z)
                p16 = jnp.exp(z)
                plsc.store_scatter(pbuf, [iota_ka + j], p16, mask=lane_lt8)
                plsc.addupdate_scatter(accP, [dv16 * 8 + iota], p16,
                                       mask=lane_lt8)
                plsc.addupdate_scatter(accWC, [dv16 * 2 + iota],
                                       jnp.where(iota == 0, wsp, 1.0),
                                       mask=lane_lt2)
                return 0
            lax.fori_loop(0, KA, edge_grp, 0)
            for h in range(H):
                pltpu.sync_copy(pbuf.at[pl.ds(h * KA, KA)],
                                p_hbm.at[pl.ds(h * E + base, KA)])
            return 0
        lax.fori_loop(0, EC // KA, chunk_body, 0)
        pltpu.sync_copy(accP, part_hbm.at[wid, pl.ds(0, N * 8)])
        pltpu.sync_copy(accWC, part_hbm.at[wid, pl.ds(N * 8, N * 2)])

    return sa1(src, dst, w, tabS, tabD, c1arr)


def _reduce_partials(part, mprow, span):
    mesh = plsc.VectorSubcoreMesh(core_axis_name="c", subcore_axis_name="s")

    @functools.partial(
        pl.kernel,
        out_type=jax.ShapeDtypeStruct((NW * span,), jnp.float32),
        mesh=mesh,
        compiler_params=pltpu.CompilerParams(use_tc_tiling_on_sc=False, needs_layout_passes=False),
        scratch_types=[
            pltpu.VMEM((span,), jnp.float32),
            pltpu.VMEM((span,), jnp.float32),
        ],
    )
    def red_k(part_hbm, out_hbm, buf, accv):
        wid = lax.axis_index("s") * NC + lax.axis_index("c")
        off = wid * span
        zero16 = jnp.zeros((16,), jnp.float32)

        def zbody(i, _):
            accv[pl.ds(i * 16, 16)] = zero16
            return 0
        lax.fori_loop(0, span // 16, zbody, 0)

        def tbody(t, _):
            pltpu.sync_copy(part_hbm.at[pl.ds(t * mprow + off, span)], buf)

            def abody(i, _):
                sl = pl.ds(i * 16, 16)
                accv[sl] = accv[sl] + buf[sl]
                return 0
            lax.fori_loop(0, span // 16, abody, 0)
            return 0
        lax.fori_loop(0, NW, tbody, 0)
        pltpu.sync_copy(accv, out_hbm.at[pl.ds(off, span)])

    return red_k(part)


def _sb1(src, dst, p1, h1v):
    mesh = plsc.VectorSubcoreMesh(core_axis_name="c", subcore_axis_name="s")

    @functools.partial(
        pl.kernel,
        out_type=jax.ShapeDtypeStruct((2, NHALF, NW, 16), jnp.float32),
        mesh=mesh,
        compiler_params=pltpu.CompilerParams(use_tc_tiling_on_sc=False, needs_layout_passes=False),
        scratch_types=[
            pltpu.VMEM((KB,), jnp.int32),
            pltpu.VMEM((KB,), jnp.int32),
            pltpu.VMEM((KB,), jnp.float32),
            pltpu.VMEM((KB,), jnp.float32),
            pltpu.VMEM((KB,), jnp.int32),
            pltpu.VMEM((KB,), jnp.int32),
            pltpu.VMEM((KB,), jnp.int32),
            pltpu.VMEM((KB,), jnp.int32),
            pltpu.VMEM((KB, 16), jnp.float32),
            pltpu.VMEM((KB, 16), jnp.float32),
            pltpu.VMEM((NACC, 16), jnp.float32),
            pltpu.SemaphoreType.DMA,
            pltpu.SemaphoreType.DMA,
            pltpu.SemaphoreType.DMA,
            pltpu.SemaphoreType.DMA,
            pltpu.SemaphoreType.DMA,
        ],
    )
    def sb1(src_hbm, dst_hbm, p_hbm, h1_hbm, out_hbm,
            src_v, dst_v, p0, p1_, idx0, idx1, dl0, dl1, rows0, rows1,
            acc, sem0, sem1, semA, semB, semC):
        wid = lax.axis_index("s") * NC + lax.axis_index("c")
        head = wid // 4
        iota = _iota16()
        zero16 = jnp.zeros((16,), jnp.float32)
        NCH = E // KB

        for half in range(2):
            def zbody(i, _):
                acc[i] = zero16
                return 0
            lax.fori_loop(0, NACC, zbody, 0)

            def load_prep(ci, p_b, idx_b, dl_b):
                base = ci * KB
                da = pltpu.async_copy(src_hbm.at[pl.ds(base, KB)], src_v, semA)
                db = pltpu.async_copy(dst_hbm.at[pl.ds(base, KB)], dst_v, semB)
                dc = pltpu.async_copy(p_hbm.at[pl.ds(head * E + base, KB)],
                                      p_b, semC)
                da.wait()
                db.wait()
                dc.wait()

                def prep(g, _):
                    sl = pl.ds(g * 16, 16)
                    sv = src_v[sl]
                    idx_b[sl] = sv * NW + wid
                    dv = dst_v[sl] - half * NHALF
                    m = (dv >= 0) & (dv < NHALF)
                    dl_b[sl] = jnp.where(m, dv, NHALF)
                    return 0
                lax.fori_loop(0, KB // 16, prep, 0)

            def compute(rows_b, p_b, dl_b):
                def edge_grp(g, _):
                    sl = pl.ds(g * 16, 16)
                    pvec = p_b[sl]
                    dlvec = dl_b[sl]
                    for u in range(16):
                        j = g * 16 + u
                        ix = jnp.full((16,), u, jnp.int32)
                        psp = pvec.at[ix].get(mode="promise_in_bounds")
                        dsp = dlvec.at[ix].get(mode="promise_in_bounds")
                        plsc.addupdate_scatter(acc, [dsp, iota],
                                               rows_b[j] * psp)
                    return 0
                lax.fori_loop(0, KB // 16, edge_grp, 0)

            load_prep(0, p0, idx0, dl0)
            pltpu.async_copy(h1_hbm.at[idx0], rows0, sem0)

            def pair(k, _):
                c0 = 2 * k
                load_prep(c0 + 1, p1_, idx1, dl1)
                pltpu.async_copy(h1_hbm.at[idx1], rows1, sem1)
                pltpu.make_async_copy(h1_hbm.at[idx0], rows0, sem0).wait()
                compute(rows0, p0, dl0)
                load_prep(c0 + 2, p0, idx0, dl0)
                pltpu.async_copy(h1_hbm.at[idx0], rows0, sem0)
                pltpu.make_async_copy(h1_hbm.at[idx1], rows1, sem1).wait()
                compute(rows1, p1_, dl1)
                return 0
            lax.fori_loop(0, NCH // 2, pair, 0)
            pltpu.make_async_copy(h1_hbm.at[idx0], rows0, sem0).wait()
            pltpu.sync_copy(acc.at[pl.ds(0, NHALF), :],
                            out_hbm.at[half, :, wid, :])

    return sb1(src, dst, p1, h1v)


def _t2_body(acc_ref, psum_ref, wsum_ref, cnt_ref, tabS_ref, tabD_ref,
             c_ref, h_ref, b_ref, w2_ref,
             asrc2_ref, adst2_ref, we2_ref, ae2_ref,
             h2_ref, tab2S_ref, tab2D_ref, c2_ref):
    psum = psum_ref[...]
    la = wsum_ref[...] / jnp.maximum(cnt_ref[...], 1.0)
    as1 = tabS_ref[...][:, 0:8]
    ad1 = tabD_ref[...][:, 0:8]
    c1 = c_ref[...][:, 0:8]
    z = as1 + ad1 + la * c1
    z = jnp.maximum(z, 0.2 * z)
    pself = jnp.exp(z)
    ssum = psum + pself + 1e-16
    h1b = h_ref[...]
    accb = acc_ref[...]
    rrow = lax.broadcasted_iota(jnp.int32, (H, F1), 0)
    rcol = lax.broadcasted_iota(jnp.int32, (H, F1), 1)
    R = jnp.where(rrow == (rcol // HID), 1.0, 0.0)
    pw = jnp.dot(pself, R, preferred_element_type=jnp.float32)
    sw = jnp.dot(ssum, R, preferred_element_type=jnp.float32)
    h1o = (accb + pw * h1b) / sw + b_ref[...]
    act = jnp.where(h1o > 0, h1o, jnp.exp(jnp.minimum(h1o, 0.0)) - 1.0)
    h2 = jnp.dot(act, w2_ref[...], preferred_element_type=jnp.float32)
    h2_ref[...] = h2
    as2 = jnp.sum(h2 * asrc2_ref[...], axis=1, keepdims=True)
    ad2 = jnp.sum(h2 * adst2_ref[...], axis=1, keepdims=True)
    ones16 = jnp.ones((1, 16), jnp.float32)
    tab2S_ref[...] = as2 * ones16
    tab2D_ref[...] = ad2 * ones16
    c2 = jnp.sum(we2_ref[...] * ae2_ref[...], axis=1, keepdims=True)
    c2_ref[...] = c2 * ones16


def _t2(acc1, psum, wsum, cnt, tabS, tabD, c1arr, h1, b1, W2,
        a_src2, a_dst2, We2, ae2):
    return pl.pallas_call(
        _t2_body,
        grid=(N // BN,),
        in_specs=[
            pl.BlockSpec((BN, F1), lambda i: (i, 0)),
            pl.BlockSpec((BN, 8), lambda i: (i, 0)),
            pl.BlockSpec((BN, 1), lambda i: (i, 0)),
            pl.BlockSpec((BN, 1), lambda i: (i, 0)),
            pl.BlockSpec((BN, 16), lambda i: (i, 0)),
            pl.BlockSpec((BN, 16), lambda i: (i, 0)),
            pl.BlockSpec((1, 16), lambda i: (0, 0)),
            pl.BlockSpec((BN, F1), lambda i: (i, 0)),
            pl.BlockSpec((1, F1), lambda i: (0, 0)),
            pl.BlockSpec((F1, DOUT), lambda i: (0, 0)),
            pl.BlockSpec((1, DOUT), lambda i: (0, 0)),
            pl.BlockSpec((1, DOUT), lambda i: (0, 0)),
            pl.BlockSpec((1, DOUT), lambda i: (0, 0)),
            pl.BlockSpec((1, DOUT), lambda i: (0, 0)),
        ],
        out_specs=[
            pl.BlockSpec((BN, DOUT), lambda i: (i, 0)),
            pl.BlockSpec((BN, 16), lambda i: (i, 0)),
            pl.BlockSpec((BN, 16), lambda i: (i, 0)),
            pl.BlockSpec((1, 16), lambda i: (0, 0)),
        ],
        out_shape=[
            jax.ShapeDtypeStruct((N, DOUT), jnp.float32),
            jax.ShapeDtypeStruct((N, 16), jnp.float32),
            jax.ShapeDtypeStruct((N, 16), jnp.float32),
            jax.ShapeDtypeStruct((1, 16), jnp.float32),
        ],
    )(acc1, psum, wsum, cnt, tabS, tabD, c1arr, h1, b1, W2,
      a_src2, a_dst2, We2, ae2)


def _sa2(src, dst, w, tab2S, tab2D, c2arr):
    mesh = plsc.VectorSubcoreMesh(core_axis_name="c", subcore_axis_name="s")

    @functools.partial(
        pl.kernel,
        out_type=[
            jax.ShapeDtypeStruct((E + KB,), jnp.float32),
            jax.ShapeDtypeStruct((NW, NP16, 16), jnp.float32),
        ],
        mesh=mesh,
        compiler_params=pltpu.CompilerParams(use_tc_tiling_on_sc=False, needs_layout_passes=False),
        scratch_types=[
            pltpu.VMEM((KA,), jnp.int32),
            pltpu.VMEM((KA,), jnp.int32),
            pltpu.VMEM((KA,), jnp.float32),
            pltpu.VMEM((KA, 16), jnp.float32),
            pltpu.VMEM((KA, 16), jnp.float32),
            pltpu.VMEM((KA,), jnp.float32),
            pltpu.VMEM((NP16, 16), jnp.float32),
            pltpu.VMEM((16,), jnp.float32),
            pltpu.SemaphoreType.DMA,
            pltpu.SemaphoreType.DMA,
        ],
    )
    def sa2(src_hbm, dst_hbm, w_hbm, tabS_hbm, tabD_hbm, c2_hbm,
            p_hbm, part_hbm,
            src_v, dst_v, w_v, rowsS, rowsD, p_v, accS, c2_v,
            sem1, sem2):
        wid = lax.axis_index("s") * NC + lax.axis_index("c")
        iota = _iota16()
        zero16 = jnp.zeros((16,), jnp.float32)
        izero = jnp.zeros((16,), jnp.int32)
        lane0 = iota == 0

        pltpu.sync_copy(c2_hbm, c2_v)
        c2v = c2_v[...]

        def zS(i, _):
            accS[i] = zero16
            return 0
        lax.fori_loop(0, NP16, zS, 0)

        base0 = wid * EC

        def chunk_body(ci, _):
            base = base0 + ci * KA
            pltpu.sync_copy(src_hbm.at[pl.ds(base, KA)], src_v)
            pltpu.sync_copy(dst_hbm.at[pl.ds(base, KA)], dst_v)
            pltpu.sync_copy(w_hbm.at[pl.ds(base, KA)], w_v)
            d1 = pltpu.async_copy(tabS_hbm.at[src_v], rowsS, sem1)
            d2 = pltpu.async_copy(tabD_hbm.at[dst_v], rowsD, sem2)
            d1.wait()
            d2.wait()

            def edge_grp(g, _):
                sl = pl.ds(g * 16, 16)
                dvec = dst_v[sl]
                wvec = w_v[sl]
                for u in range(16):
                    j = g * 16 + u
                    dv = dvec[u]
                    z = rowsS[j] + rowsD[j] + wvec[u] * c2v
                    z = jnp.maximum(z, 0.2 * z)
                    p16 = jnp.exp(z)
                    plsc.store_scatter(p_v, [izero + j], p16, mask=lane0)
                    plsc.addupdate(accS.at[dv // 16],
                                   jnp.where(iota == (dv % 16), p16, 0.0))
                return 0
            lax.fori_loop(0, KA // 16, edge_grp, 0)
            pltpu.sync_copy(p_v, p_hbm.at[pl.ds(base, KA)])
            return 0
        lax.fori_loop(0, EC // KA, chunk_body, 0)
        pltpu.sync_copy(accS, part_hbm.at[wid])

    return sa2(src, dst, w, tab2S, tab2D, c2arr)


def _sb2(src, dst, p2, h2v):
    mesh = plsc.VectorSubcoreMesh(core_axis_name="c", subcore_axis_name="s")
    EQ = E // 4

    @functools.partial(
        pl.kernel,
        out_type=jax.ShapeDtypeStruct((2, NHALF, 4, 8, 16), jnp.float32),
        mesh=mesh,
        compiler_params=pltpu.CompilerParams(use_tc_tiling_on_sc=False, needs_layout_passes=False),
        scratch_types=[
            pltpu.VMEM((KB,), jnp.int32),
            pltpu.VMEM((KB,), jnp.int32),
            pltpu.VMEM((KB,), jnp.float32),
            pltpu.VMEM((KB,), jnp.float32),
            pltpu.VMEM((KB,), jnp.int32),
            pltpu.VMEM((KB,), jnp.int32),
            pltpu.VMEM((KB,), jnp.int32),
            pltpu.VMEM((KB,), jnp.int32),
            pltpu.VMEM((KB, 16), jnp.float32),
            pltpu.VMEM((KB, 16), jnp.float32),
            pltpu.VMEM((NACC, 16), jnp.float32),
            pltpu.SemaphoreType.DMA,
            pltpu.SemaphoreType.DMA,
            pltpu.SemaphoreType.DMA,
            pltpu.SemaphoreType.DMA,
            pltpu.SemaphoreType.DMA,
        ],
    )
    def sb2(src_hbm, dst_hbm, p_hbm, h2_hbm, out_hbm,
            src_v, dst_v, p0, p1_, idx0, idx1, dl0, dl1, rows0, rows1,
            acc, sem0, sem1, semA, semB, semC):
        wid = lax.axis_index("s") * NC + lax.axis_index("c")
        q = wid // 8
        col = wid % 8
        iota = _iota16()
        zero16 = jnp.zeros((16,), jnp.float32)
        ebase = q * EQ
        NCH = EQ // KB

        for half in range(2):
            def zbody(i, _):
                acc[i] = zero16
                return 0
            lax.fori_loop(0, NACC, zbody, 0)

            def load_prep(ci, p_b, idx_b, dl_b):
                base = ebase + ci * KB
                da = pltpu.async_copy(src_hbm.at[pl.ds(base, KB)], src_v, semA)
                db = pltpu.async_copy(dst_hbm.at[pl.ds(base, KB)], dst_v, semB)
                dc = pltpu.async_copy(p_hbm.at[pl.ds(base, KB)], p_b, semC)
                da.wait()
                db.wait()
                dc.wait()

                def prep(g, _):
                    sl = pl.ds(g * 16, 16)
                    sv = src_v[sl]
                    idx_b[sl] = sv * 8 + col
                    dv = dst_v[sl] - half * NHALF
                    m = (dv >= 0) & (dv < NHALF)
                    dl_b[sl] = jnp.where(m, dv, NHALF)
                    return 0
                lax.fori_loop(0, KB // 16, prep, 0)

            def compute(rows_b, p_b, dl_b):
                def edge_grp(g, _):
                    sl = pl.ds(g * 16, 16)
                    pvec = p_b[sl]
                    dlvec = dl_b[sl]
                    for u in range(16):
                        j = g * 16 + u
                        ix = jnp.full((16,), u, jnp.int32)
                        psp = pvec.at[ix].get(mode="promise_in_bounds")
                        dsp = dlvec.at[ix].get(mode="promise_in_bounds")
                        plsc.addupdate_scatter(acc, [dsp, iota],
                                               rows_b[j] * psp)
                    return 0
                lax.fori_loop(0, KB // 16, edge_grp, 0)

            load_prep(0, p0, idx0, dl0)
            pltpu.async_copy(h2_hbm.at[idx0], rows0, sem0)

            def pair(k, _):
                c0 = 2 * k
                load_prep(c0 + 1, p1_, idx1, dl1)
                pltpu.async_copy(h2_hbm.at[idx1], rows1, sem1)
                pltpu.make_async_copy(h2_hbm.at[idx0], rows0, sem0).wait()
                compute(rows0, p0, dl0)
                load_prep(c0 + 2, p0, idx0, dl0)
                pltpu.async_copy(h2_hbm.at[idx0], rows0, sem0)
                pltpu.make_async_copy(h2_hbm.at[idx1], rows1, sem1).wait()
                compute(rows1, p1_, dl1)
                return 0
            lax.fori_loop(0, NCH // 2, pair, 0)
            pltpu.make_async_copy(h2_hbm.at[idx0], rows0, sem0).wait()
            pltpu.sync_copy(acc.at[pl.ds(0, NHALF), :],
                            out_hbm.at[half, :, q, col, :])

    return sb2(src, dst, p2, h2v)


def _t3_body(acc_ref, part_ref, tabS_ref, tabD_ref, wsum_ref, cnt_ref,
             c2_ref, h2_ref, b2_ref, out_ref):
    accsum = jnp.sum(acc_ref[...], axis=1)
    ssum = part_ref[...]
    as2 = tabS_ref[...][:, 0:1]
    ad2 = tabD_ref[...][:, 0:1]
    la = wsum_ref[...] / jnp.maximum(cnt_ref[...], 1.0)
    c2 = c2_ref[...][:, 0:1]
    z = as2 + ad2 + la * c2
    z = jnp.maximum(z, 0.2 * z)
    ps = jnp.exp(z)
    h2b = h2_ref[...]
    out = (accsum + ps * h2b) / (ssum + ps + 1e-16)
    out_ref[...] = out + b2_ref[...]


def _t3(acc2, part2, tab2S, tab2D, wsum, cnt, c2arr, h2, b2):
    return pl.pallas_call(
        _t3_body,
        grid=(N // BN,),
        in_specs=[
            pl.BlockSpec((BN, 4, DOUT), lambda i: (i, 0, 0)),
            pl.BlockSpec((BN, 1), lambda i: (i, 0)),
            pl.BlockSpec((BN, 16), lambda i: (i, 0)),
            pl.BlockSpec((BN, 16), lambda i: (i, 0)),
            pl.BlockSpec((BN, 1), lambda i: (i, 0)),
            pl.BlockSpec((BN, 1), lambda i: (i, 0)),
            pl.BlockSpec((1, 16), lambda i: (0, 0)),
            pl.BlockSpec((BN, DOUT), lambda i: (i, 0)),
            pl.BlockSpec((1, DOUT), lambda i: (0, 0)),
        ],
        out_specs=pl.BlockSpec((BN, DOUT), lambda i: (i, 0)),
        out_shape=jax.ShapeDtypeStruct((N, DOUT), jnp.float32),
    )(acc2, part2, tab2S, tab2D, wsum, cnt, c2arr, h2, b2)


def kernel(x, edge_index, edge_weight, W1, a_src1, a_dst1, We1, ae1, b1,
           W2, a_src2, a_dst2, We2, ae2, b2):
    src = edge_index[0]
    dst = edge_index[1]
    pad = jnp.zeros((KB,), dtype=edge_index.dtype)
    srcp = jnp.concatenate([src, pad])
    dstp = jnp.concatenate([dst, pad])
    h1, tabS, tabD, c1arr = _t1(x, W1, a_src1.reshape(1, F1),
                                a_dst1.reshape(1, F1), We1, ae1.reshape(1, F1))
    p1, part1 = _sa1(src, dst, edge_weight, tabS, tabD, c1arr.reshape(16))
    red1 = _reduce_partials(part1.reshape(NW * 32 * SPAN1), 32 * SPAN1, SPAN1)
    psum = red1[:N * 8].reshape(N, 8)
    wc = red1[N * 8:N * 10].reshape(N, 2)
    wsum = wc[:, 0:1]
    cnt = wc[:, 1:2]
    acc1 = _sb1(srcp, dstp, p1, h1.reshape(N * NW, 16))
    h2, tab2S, tab2D, c2arr = _t2(
        acc1.reshape(N, F1), psum, wsum, cnt, tabS, tabD, c1arr, h1,
        b1.reshape(1, F1), W2, a_src2.reshape(1, DOUT), a_dst2.reshape(1, DOUT),
        We2, ae2.reshape(1, DOUT))
    p2, part2 = _sa2(src, dst, edge_weight, tab2S, tab2D, c2arr.reshape(16))
    red2 = _reduce_partials(part2.reshape(NW * M2), M2, SPAN2)
    ssum2 = red2[:N].reshape(N, 1)
    acc2 = _sb2(srcp, dstp, p2, h2.reshape(N * 8, 16))
    out = _t3(acc2.reshape(N, 4, DOUT), ssum2, tab2S, tab2D, wsum, cnt,
              c2arr, h2, b2.reshape(1, DOUT))
    return out
